# baseline (speedup 1.0000x reference)
import functools

import numpy as np
import jax
import jax.numpy as jnp
from jax import lax
from jax.experimental import pallas as pl
from jax.experimental.pallas import tpu as pltpu
from jax.experimental.pallas import tpu_sc as plsc

_COV = [0.2, 0.31, 0.28, 1.28, 0.96, 0.84, 0.76, 0.71, 0.66, 0.57, 0.58, 1.66, 1.41, 1.21,
        1.11, 1.07, 1.05, 1.02, 1.06, 2.03, 1.76, 1.70, 1.60, 1.53, 1.39, 1.39, 1.32, 1.26,
        1.24, 1.32, 1.22, 1.22, 1.20, 1.19, 1.20, 1.20, 1.16, 2.20, 1.95, 1.90, 1.75, 1.64,
        1.54, 1.47, 1.46, 1.42, 1.39, 1.45, 1.44, 1.42, 1.39, 1.39, 1.38, 1.39, 1.40, 2.44,
        2.15, 2.07, 2.04, 2.03, 2.01, 1.99, 1.98, 1.98, 1.96, 1.94, 1.92, 1.92, 1.89, 1.90,
        1.87, 1.87, 1.75, 1.70, 1.62, 1.51, 1.44, 1.41, 1.36, 1.36, 1.32, 1.45, 1.46, 1.48,
        1.40, 1.50, 1.50, 2.60, 2.21, 2.15, 2.06, 2.00, 1.96, 1.90, 1.87, 1.80, 1.69]
_COV = _COV + [0.2] * (119 - len(_COV))
_COV_TABLE = np.asarray(_COV, dtype=np.float32)

_NW = 32
_CHUNK = 2048


def _node_radius_tc(attrs_t, cov_species):
    n_species = cov_species.shape[1]

    def body(attrs_ref, cov_ref, out_ref):
        am = jnp.argmax(attrs_ref[...], axis=0, keepdims=True)
        r = jnp.zeros(am.shape, jnp.float32) + cov_ref[0:1, 0:1]
        for sp in range(1, n_species):
            r = jnp.where(am == sp, cov_ref[0:1, sp:sp + 1], r)
        out_ref[...] = r

    return pl.pallas_call(
        body,
        out_shape=jax.ShapeDtypeStruct((1, attrs_t.shape[1]), jnp.float32),
    )(attrs_t, cov_species)


def _edge_transform_sc(node_r, ei3, x_flat, n_nodes, n_edges):
    nblk = _CHUNK // 128
    nchunks = n_edges // _CHUNK
    base_per_w = nchunks // _NW
    extra = nchunks % _NW
    gmax = base_per_w + (1 if extra else 0)
    mesh = plsc.VectorSubcoreMesh(core_axis_name="c", subcore_axis_name="s")

    @functools.partial(
        pl.kernel, mesh=mesh,
        out_type=jax.ShapeDtypeStruct((n_edges,), jnp.float32),
        compiler_params=pltpu.CompilerParams(
            needs_layout_passes=False, use_tc_tiling_on_sc=False),
        scratch_types=[
            pltpu.VMEM((n_nodes,), jnp.float32),
            pltpu.VMEM((2 * _CHUNK,), jnp.int32),
            pltpu.VMEM((2 * _CHUNK,), jnp.int32),
            pltpu.VMEM((_CHUNK,), jnp.float32), pltpu.VMEM((_CHUNK,), jnp.float32),
            pltpu.VMEM((_CHUNK,), jnp.float32), pltpu.VMEM((_CHUNK,), jnp.float32),
            pltpu.SemaphoreType.DMA, pltpu.SemaphoreType.DMA,
            pltpu.SemaphoreType.DMA, pltpu.SemaphoreType.DMA,
            pltpu.SemaphoreType.DMA,
        ],
    )
    def edge_kernel(noder_hbm, ei_hbm, x_hbm, out_hbm, table,
                    e0, e1, x0, x1, o0, o1,
                    si0, si1, so0, so1, semt):
        cid = lax.axis_index("c")
        sid = lax.axis_index("s")
        wid = sid * 2 + cid
        cstart = wid * base_per_w + jnp.minimum(wid, extra)

        def cidx(gi):
            return jnp.minimum(cstart + gi, nchunks - 1)

        slots = ((e0, x0, o0, si0, so0), (e1, x1, o1, si1, so1))

        def issue_in(b, ci):
            eb, xb, _, si, _ = slots[b]
            pltpu.async_copy(ei_hbm.at[pl.ds(ci * 2 * _CHUNK, 2 * _CHUNK)], eb, si)
            pltpu.async_copy(x_hbm.at[pl.ds(ci * _CHUNK, _CHUNK)], xb, si)

        def wait_in(b):
            eb, xb, _, si, _ = slots[b]
            pltpu.make_async_copy(ei_hbm.at[pl.ds(0, 2 * _CHUNK)], eb, si).wait()
            pltpu.make_async_copy(x_hbm.at[pl.ds(0, _CHUNK)], xb, si).wait()

        def wait_out(b):
            ob, so = slots[b][2], slots[b][4]
            pltpu.make_async_copy(ob, out_hbm.at[pl.ds(0, _CHUNK)], so).wait()

        tdesc = pltpu.async_copy(noder_hbm, table, semt)
        issue_in(0, cidx(0))
        issue_in(1, cidx(1))
        tdesc.wait()

        def outer(g, carry):
            for b in range(2):
                gi = g * 2 + b
                ci = cidx(gi)
                eb, xb, ob, si, so = slots[b]
                wait_in(b)

                @pl.when(g > 0)
                def _():
                    wait_out(b)

                @plsc.parallel_loop(0, _CHUNK // 16, unroll=8)
                def vec_body(j):
                    soff = j * 16 + (j >> 3) * 128
                    sv = eb[pl.ds(soff, 16)]
                    rv = eb[pl.ds(soff + 128, 16)]
                    r0 = plsc.load_gather(table, [sv]) + plsc.load_gather(table, [rv])
                    dsx = pl.ds(j * 16, 16)
                    xx = xb[dsx]
                    e = jnp.exp((100.0 / 7.0) - (96.0 / 7.0) * (xx / r0))
                    p0 = 0.75 * r0
                    ob[dsx] = (p0 * e + xx) / (1.0 + e)

                pltpu.async_copy(ob, out_hbm.at[pl.ds(ci * _CHUNK, _CHUNK)], so)
                issue_in(b, cidx(gi + 2))
            return carry

        lax.fori_loop(0, (gmax + 1) // 2, outer, 0)
        wait_in(0)
        wait_in(1)
        wait_out(0)
        wait_out(1)

    return edge_kernel(node_r, ei3, x_flat)


def kernel(x, node_attrs, edge_index, atomic_numbers):
    n_edges = x.shape[0]
    n_nodes, n_species = node_attrs.shape

    cov = jnp.asarray(_COV_TABLE)
    cov_species = cov[jnp.clip(atomic_numbers.astype(jnp.int32), 0, 118)]

    node_r = _node_radius_tc(node_attrs.T, cov_species.reshape(1, n_species))
    node_r = node_r.reshape(n_nodes)

    ei3 = (edge_index.astype(jnp.int32)
           .reshape(2, n_edges // 128, 128).transpose(1, 0, 2)
           .reshape(2 * n_edges))
    x_flat = x.reshape(n_edges)
    out = _edge_transform_sc(node_r, ei3, x_flat, n_nodes, n_edges)
    return out.reshape(n_edges, 1)


# speedup vs baseline: 1.0259x; 1.0259x over previous
import functools

import numpy as np
import jax
import jax.numpy as jnp
from jax import lax
from jax.experimental import pallas as pl
from jax.experimental.pallas import tpu as pltpu
from jax.experimental.pallas import tpu_sc as plsc

_COV = [0.2, 0.31, 0.28, 1.28, 0.96, 0.84, 0.76, 0.71, 0.66, 0.57, 0.58, 1.66, 1.41, 1.21,
        1.11, 1.07, 1.05, 1.02, 1.06, 2.03, 1.76, 1.70, 1.60, 1.53, 1.39, 1.39, 1.32, 1.26,
        1.24, 1.32, 1.22, 1.22, 1.20, 1.19, 1.20, 1.20, 1.16, 2.20, 1.95, 1.90, 1.75, 1.64,
        1.54, 1.47, 1.46, 1.42, 1.39, 1.45, 1.44, 1.42, 1.39, 1.39, 1.38, 1.39, 1.40, 2.44,
        2.15, 2.07, 2.04, 2.03, 2.01, 1.99, 1.98, 1.98, 1.96, 1.94, 1.92, 1.92, 1.89, 1.90,
        1.87, 1.87, 1.75, 1.70, 1.62, 1.51, 1.44, 1.41, 1.36, 1.36, 1.32, 1.45, 1.46, 1.48,
        1.40, 1.50, 1.50, 2.60, 2.21, 2.15, 2.06, 2.00, 1.96, 1.90, 1.87, 1.80, 1.69]
_COV = _COV + [0.2] * (119 - len(_COV))
_COV_TABLE = np.asarray(_COV, dtype=np.float32)

_NW = 32
_CHUNK = 2048


def _node_radius_tc(attrs_t, cov_species):
    n_species = cov_species.shape[1]

    def body(attrs_ref, cov_ref, out_ref):
        best = attrs_ref[0:1, :]
        r = jnp.zeros_like(best) + cov_ref[0:1, 0:1]
        for sp in range(1, n_species):
            v = attrs_ref[sp:sp + 1, :]
            m = v > best
            best = jnp.where(m, v, best)
            r = jnp.where(m, cov_ref[0:1, sp:sp + 1], r)
        out_ref[...] = r

    return pl.pallas_call(
        body,
        out_shape=jax.ShapeDtypeStruct((1, attrs_t.shape[1]), jnp.float32),
    )(attrs_t, cov_species)


def _edge_transform_sc(node_r, ei3, x_flat, n_nodes, n_edges):
    nblk = _CHUNK // 128
    nchunks = n_edges // _CHUNK
    base_per_w = nchunks // _NW
    extra = nchunks % _NW
    gmax = base_per_w + (1 if extra else 0)
    mesh = plsc.VectorSubcoreMesh(core_axis_name="c", subcore_axis_name="s")

    @functools.partial(
        pl.kernel, mesh=mesh,
        out_type=jax.ShapeDtypeStruct((n_edges,), jnp.float32),
        compiler_params=pltpu.CompilerParams(
            needs_layout_passes=False, use_tc_tiling_on_sc=False),
        scratch_types=[
            pltpu.VMEM((n_nodes,), jnp.float32),
            pltpu.VMEM((2 * _CHUNK,), jnp.int32),
            pltpu.VMEM((2 * _CHUNK,), jnp.int32),
            pltpu.VMEM((_CHUNK,), jnp.float32), pltpu.VMEM((_CHUNK,), jnp.float32),
            pltpu.VMEM((_CHUNK,), jnp.float32), pltpu.VMEM((_CHUNK,), jnp.float32),
            pltpu.SemaphoreType.DMA, pltpu.SemaphoreType.DMA,
            pltpu.SemaphoreType.DMA, pltpu.SemaphoreType.DMA,
            pltpu.SemaphoreType.DMA,
        ],
    )
    def edge_kernel(noder_hbm, ei_hbm, x_hbm, out_hbm, table,
                    e0, e1, x0, x1, o0, o1,
                    si0, si1, so0, so1, semt):
        cid = lax.axis_index("c")
        sid = lax.axis_index("s")
        wid = sid * 2 + cid
        cstart = wid * base_per_w + jnp.minimum(wid, extra)

        def cidx(gi):
            return jnp.minimum(cstart + gi, nchunks - 1)

        slots = ((e0, x0, o0, si0, so0), (e1, x1, o1, si1, so1))

        def issue_in(b, ci):
            eb, xb, _, si, _ = slots[b]
            pltpu.async_copy(ei_hbm.at[pl.ds(ci * 2 * _CHUNK, 2 * _CHUNK)], eb, si)
            pltpu.async_copy(x_hbm.at[pl.ds(ci * _CHUNK, _CHUNK)], xb, si)

        def wait_in(b):
            eb, xb, _, si, _ = slots[b]
            pltpu.make_async_copy(ei_hbm.at[pl.ds(0, 2 * _CHUNK)], eb, si).wait()
            pltpu.make_async_copy(x_hbm.at[pl.ds(0, _CHUNK)], xb, si).wait()

        def wait_out(b):
            ob, so = slots[b][2], slots[b][4]
            pltpu.make_async_copy(ob, out_hbm.at[pl.ds(0, _CHUNK)], so).wait()

        tdesc = pltpu.async_copy(noder_hbm, table, semt)
        issue_in(0, cidx(0))
        issue_in(1, cidx(1))
        tdesc.wait()

        def outer(g, carry):
            for b in range(2):
                gi = g * 2 + b
                ci = cidx(gi)
                eb, xb, ob, si, so = slots[b]
                wait_in(b)

                @pl.when(g > 0)
                def _():
                    wait_out(b)

                @plsc.parallel_loop(0, _CHUNK // 16, unroll=8)
                def vec_body(j):
                    soff = j * 16 + (j >> 3) * 128
                    sv = eb[pl.ds(soff, 16)]
                    rv = eb[pl.ds(soff + 128, 16)]
                    r0 = plsc.load_gather(table, [sv]) + plsc.load_gather(table, [rv])
                    dsx = pl.ds(j * 16, 16)
                    xx = xb[dsx]
                    e = jnp.exp((100.0 / 7.0) - (96.0 / 7.0) * (xx / r0))
                    p0 = 0.75 * r0
                    ob[dsx] = (p0 * e + xx) / (1.0 + e)

                pltpu.async_copy(ob, out_hbm.at[pl.ds(ci * _CHUNK, _CHUNK)], so)
                issue_in(b, cidx(gi + 2))
            return carry

        lax.fori_loop(0, (gmax + 1) // 2, outer, 0)
        wait_in(0)
        wait_in(1)
        wait_out(0)
        wait_out(1)

    return edge_kernel(node_r, ei3, x_flat)


def kernel(x, node_attrs, edge_index, atomic_numbers):
    n_edges = x.shape[0]
    n_nodes, n_species = node_attrs.shape

    cov = jnp.asarray(_COV_TABLE)
    cov_species = cov[jnp.clip(atomic_numbers.astype(jnp.int32), 0, 118)]

    node_r = _node_radius_tc(node_attrs.T, cov_species.reshape(1, n_species))
    node_r = node_r.reshape(n_nodes)

    ei3 = (edge_index.astype(jnp.int32)
           .reshape(2, n_edges // 128, 128).transpose(1, 0, 2)
           .reshape(2 * n_edges))
    x_flat = x.reshape(n_edges)
    out = _edge_transform_sc(node_r, ei3, x_flat, n_nodes, n_edges)
    return out.reshape(n_edges, 1)
